# baseline (speedup 1.0000x reference)
import functools

import jax
import jax.numpy as jnp
from jax import lax
from jax.experimental import pallas as pl
from jax.experimental.pallas import tpu as pltpu
from jax.experimental.pallas import tpu_sc as plsc

_NC = 2
_NS = 16
_NW = _NC * _NS


def _gather_fn(N, D, C, NBUF):
    n_chunks = N // (_NW * C)
    b_per_w = N // _NW
    mesh = plsc.VectorSubcoreMesh(core_axis_name="c", subcore_axis_name="s")

    @functools.partial(
        pl.kernel,
        mesh=mesh,
        out_type=jax.ShapeDtypeStruct((N, D), jnp.float32),
        scratch_types=[
            pltpu.VMEM((b_per_w,), jnp.int32),
            pltpu.VMEM((NBUF, C, D), jnp.float32),
        ]
        + [pltpu.SemaphoreType.DMA] * (2 * NBUF),
        compiler_params=pltpu.CompilerParams(use_tc_tiling_on_sc=False),
    )
    def k(table_hbm, idx_hbm, out_hbm, idx_v, rows_v, *sems):
        g_sems = sems[:NBUF]
        s_sems = sems[NBUF:]
        wid = lax.axis_index("s") * _NC + lax.axis_index("c")
        base = wid * b_per_w
        pltpu.sync_copy(idx_hbm.at[pl.ds(base, b_per_w)], idx_v)

        gathers = [None] * NBUF
        stores = [None] * NBUF
        for j in range(min(NBUF - 1, n_chunks)):
            gathers[j] = pltpu.async_copy(
                table_hbm.at[idx_v.at[pl.ds(j * C, C)]], rows_v.at[j], g_sems[j]
            )
        for i in range(n_chunks):
            b = i % NBUF
            pre = i + NBUF - 1
            if pre < n_chunks:
                pb = pre % NBUF
                if stores[pb] is not None:
                    stores[pb].wait()
                gathers[pb] = pltpu.async_copy(
                    table_hbm.at[idx_v.at[pl.ds(pre * C, C)]],
                    rows_v.at[pb],
                    g_sems[pb],
                )
            gathers[b].wait()
            stores[b] = pltpu.async_copy(
                rows_v.at[b], out_hbm.at[pl.ds(base + i * C, C)], s_sems[b]
            )
        for st in stores:
            if st is not None:
                st.wait()

    return k


_TC_CHUNK = 2048
_TC_R = 16


def _tc_gather(table, idx_tc):
    M = idx_tc.shape[0]
    V, D = table.shape
    n_pairs = M // (2 * _TC_CHUNK)
    n_groups = _TC_CHUNK // _TC_R

    def body(idx_hbm, table_hbm, out_hbm, idx_s, isem0, isem1, dsems):
        def idx_copy(chunk, buf, sem):
            return pltpu.make_async_copy(
                idx_hbm.at[pl.ds(chunk * _TC_CHUNK, _TC_CHUNK)],
                idx_s.at[buf],
                sem,
            )

        def rows(chunk_base_row, buf, skip_first_wait):
            def g_body(g, _):
                for r in range(_TC_R):
                    j = g * _TC_R + r
                    v = idx_s[buf, j]
                    cp = pltpu.make_async_copy(
                        table_hbm.at[pl.ds(v, 1)],
                        out_hbm.at[pl.ds(chunk_base_row + j, 1)],
                        dsems.at[r],
                    )

                    @pl.when(jnp.logical_or(jnp.logical_not(skip_first_wait), g > 0))
                    def _():
                        cp.wait()

                    cp.start()
                return 0

            lax.fori_loop(0, n_groups, g_body, 0)

        idx_copy(0, 0, isem0).start()

        def pair(p, _):
            c0 = 2 * p
            idx_copy(c0, 0, isem0).wait()
            idx_copy(c0 + 1, 1, isem1).start()
            rows(c0 * _TC_CHUNK, 0, p == 0)
            idx_copy(c0 + 1, 1, isem1).wait()

            @pl.when(p + 1 < n_pairs)
            def _():
                idx_copy(c0 + 2, 0, isem0).start()

            rows((c0 + 1) * _TC_CHUNK, 1, jnp.bool_(False))
            return 0

        lax.fori_loop(0, n_pairs, pair, 0)

        for r in range(_TC_R):
            pltpu.make_async_copy(
                table_hbm.at[pl.ds(0, 1)], out_hbm.at[pl.ds(0, 1)], dsems.at[r]
            ).wait()

    return pl.pallas_call(
        body,
        out_shape=jax.ShapeDtypeStruct((M, D), jnp.float32),
        in_specs=[
            pl.BlockSpec(memory_space=pl.ANY),
            pl.BlockSpec(memory_space=pl.ANY),
        ],
        out_specs=pl.BlockSpec(memory_space=pl.ANY),
        scratch_shapes=[
            pltpu.SMEM((2, _TC_CHUNK), jnp.int32),
            pltpu.SemaphoreType.DMA,
            pltpu.SemaphoreType.DMA,
            pltpu.SemaphoreType.DMA((_TC_R,)),
        ],
    )(idx_tc, table)


def kernel(weight, words):
    B, H = words.shape
    V, D = weight.shape
    N = B * H
    flat = words.reshape(N).astype(jnp.int32)
    C = 1280
    NBUF = 2
    N_TC = 40960
    N_SC = N - N_TC
    sc_out = _gather_fn(N_SC, D, C, NBUF)(weight, flat[:N_SC])
    tc_out = _tc_gather(weight, flat[N_SC:])
    return jnp.concatenate([sc_out, tc_out], axis=0).reshape(B, H, D)


# speedup vs baseline: 2.7100x; 2.7100x over previous
import functools

import jax
import jax.numpy as jnp
from jax import lax
from jax.experimental import pallas as pl
from jax.experimental.pallas import tpu as pltpu
from jax.experimental.pallas import tpu_sc as plsc

_NC = 2
_NS = 16
_NW = _NC * _NS


def _gather_fn(N, D, C, NBUF):
    n_chunks = N // (_NW * C)
    b_per_w = N // _NW
    mesh = plsc.VectorSubcoreMesh(core_axis_name="c", subcore_axis_name="s")

    @functools.partial(
        pl.kernel,
        mesh=mesh,
        out_type=jax.ShapeDtypeStruct((N, D), jnp.float32),
        scratch_types=[
            pltpu.VMEM((b_per_w,), jnp.int32),
            pltpu.VMEM((NBUF, C, D), jnp.float32),
        ]
        + [pltpu.SemaphoreType.DMA] * (2 * NBUF),
        compiler_params=pltpu.CompilerParams(use_tc_tiling_on_sc=False),
    )
    def k(table_hbm, idx_hbm, out_hbm, idx_v, rows_v, *sems):
        g_sems = sems[:NBUF]
        s_sems = sems[NBUF:]
        wid = lax.axis_index("s") * _NC + lax.axis_index("c")
        base = wid * b_per_w
        pltpu.sync_copy(idx_hbm.at[pl.ds(base, b_per_w)], idx_v)

        gathers = [None] * NBUF
        stores = [None] * NBUF
        for j in range(min(NBUF - 1, n_chunks)):
            gathers[j] = pltpu.async_copy(
                table_hbm.at[idx_v.at[pl.ds(j * C, C)]], rows_v.at[j], g_sems[j]
            )
        for i in range(n_chunks):
            b = i % NBUF
            pre = i + NBUF - 1
            if pre < n_chunks:
                pb = pre % NBUF
                if stores[pb] is not None:
                    stores[pb].wait()
                gathers[pb] = pltpu.async_copy(
                    table_hbm.at[idx_v.at[pl.ds(pre * C, C)]],
                    rows_v.at[pb],
                    g_sems[pb],
                )
            gathers[b].wait()
            stores[b] = pltpu.async_copy(
                rows_v.at[b], out_hbm.at[pl.ds(base + i * C, C)], s_sems[b]
            )
        for st in stores:
            if st is not None:
                st.wait()

    return k


def kernel(weight, words):
    B, H = words.shape
    V, D = weight.shape
    N = B * H
    flat = words.reshape(N).astype(jnp.int32)
    C = 1600
    NBUF = 2
    out = _gather_fn(N, D, C, NBUF)(weight, flat)
    return out.reshape(B, H, D)
